# Initial kernel scaffold; baseline (speedup 1.0000x reference)
import functools

import jax
import jax.numpy as jnp
from jax import lax
from jax.experimental import pallas as pl
from jax.experimental.pallas import tpu as pltpu
from jax.experimental.pallas import tpu_sc as plsc

B, L = 4096, 200
TIME_VOCAB = 1440 + 2
DIS_VOCAB = 100000 + 2
SPEED_VOCAB = 1000 + 2
EMBED = 8

N = B * L
NC, NS, LANES = 2, 16, 16
NW = NC * NS
N_PER_W = N // NW
CHUNK = 2560
NCHUNK = N_PER_W // CHUNK
OFF_DIS = TIME_VOCAB
OFF_SPEED = TIME_VOCAB + DIS_VOCAB


def _body(t_hbm, d_hbm, s_hbm, w_hbm, out_hbm,
          t_v, d_v, s_v, idx_v, rows_v, sem):
    wid = lax.axis_index("s") * NC + lax.axis_index("c")
    iota3 = lax.iota(jnp.int32, LANES) * 3

    def do_chunk(ch, _):
        base = wid * N_PER_W + ch * CHUNK
        pltpu.sync_copy(t_hbm.at[pl.ds(base, CHUNK)], t_v)
        pltpu.sync_copy(d_hbm.at[pl.ds(base, CHUNK)], d_v)
        pltpu.sync_copy(s_hbm.at[pl.ds(base, CHUNK)], s_v)

        def interleave(j, _):
            tv = t_v[pl.ds(j * LANES, LANES)]
            dv = d_v[pl.ds(j * LANES, LANES)] + OFF_DIS
            sv = s_v[pl.ds(j * LANES, LANES)] + OFF_SPEED
            pos = iota3 + j * (3 * LANES)
            plsc.store_scatter(idx_v, [pos], tv)
            plsc.store_scatter(idx_v, [pos + 1], dv)
            plsc.store_scatter(idx_v, [pos + 2], sv)
            return 0

        lax.fori_loop(0, CHUNK // LANES, interleave, 0)
        pltpu.async_copy(w_hbm.at[idx_v], rows_v, sem).wait()
        pltpu.sync_copy(rows_v, out_hbm.at[pl.ds(3 * base, 3 * CHUNK)])
        return 0

    lax.fori_loop(0, NCHUNK, do_chunk, 0)


@functools.partial(jax.jit, static_argnames=())
def kernel(time, dis, speed, W_time, W_dis, W_speed):
    w_all = jnp.concatenate([W_time, W_dis, W_speed], axis=0)
    t = time.reshape(-1).astype(jnp.int32)
    d = dis.reshape(-1).astype(jnp.int32)
    s = speed.reshape(-1).astype(jnp.int32)

    mesh = plsc.VectorSubcoreMesh(core_axis_name="c", subcore_axis_name="s")
    run = pl.kernel(
        _body,
        out_type=jax.ShapeDtypeStruct((3 * N, EMBED), jnp.float32),
        mesh=mesh,
        scratch_types=[
            pltpu.VMEM((CHUNK,), jnp.int32),
            pltpu.VMEM((CHUNK,), jnp.int32),
            pltpu.VMEM((CHUNK,), jnp.int32),
            pltpu.VMEM((3 * CHUNK,), jnp.int32),
            pltpu.VMEM((3 * CHUNK, EMBED), jnp.float32),
            pltpu.SemaphoreType.DMA,
        ],
    )
    out = run(t, d, s, w_all)
    return out.reshape(B, L, 3 * EMBED)


# speedup vs baseline: 11.0479x; 11.0479x over previous
import functools

import jax
import jax.numpy as jnp
from jax import lax
from jax.experimental import pallas as pl
from jax.experimental.pallas import tpu as pltpu
from jax.experimental.pallas import tpu_sc as plsc

B, L = 4096, 200
TIME_VOCAB = 1440 + 2
DIS_VOCAB = 100000 + 2
SPEED_VOCAB = 1000 + 2
EMBED = 8

N = B * L
NC, NS, LANES = 2, 16, 16
NW = NC * NS
N_PER_W = N // NW
CHUNK = 2560
NCHUNK = N_PER_W // CHUNK
OFF_DIS = TIME_VOCAB
OFF_SPEED = TIME_VOCAB + DIS_VOCAB


def _body(t_hbm, d_hbm, s_hbm, w_hbm, out_hbm,
          t_v, d_v, s_v, idx_v, rows_v, sem):
    wid = lax.axis_index("s") * NC + lax.axis_index("c")
    iota3 = lax.iota(jnp.int32, LANES) * 3

    def do_chunk(ch, _):
        base = wid * N_PER_W + ch * CHUNK
        pltpu.sync_copy(t_hbm.at[pl.ds(base, CHUNK)], t_v)
        pltpu.sync_copy(d_hbm.at[pl.ds(base, CHUNK)], d_v)
        pltpu.sync_copy(s_hbm.at[pl.ds(base, CHUNK)], s_v)

        def interleave(j, _):
            tv = t_v[pl.ds(j * LANES, LANES)]
            dv = d_v[pl.ds(j * LANES, LANES)] + OFF_DIS
            sv = s_v[pl.ds(j * LANES, LANES)] + OFF_SPEED
            pos = iota3 + j * (3 * LANES)
            plsc.store_scatter(idx_v, [pos], tv)
            plsc.store_scatter(idx_v, [pos + 1], dv)
            plsc.store_scatter(idx_v, [pos + 2], sv)
            return 0

        lax.fori_loop(0, CHUNK // LANES, interleave, 0)
        pltpu.async_copy(w_hbm.at[idx_v], rows_v, sem).wait()
        pltpu.sync_copy(rows_v, out_hbm.at[pl.ds(3 * base, 3 * CHUNK)])
        return 0

    lax.fori_loop(0, NCHUNK, do_chunk, 0)


@functools.partial(jax.jit, static_argnames=())
def kernel(time, dis, speed, W_time, W_dis, W_speed):
    w_all = jnp.concatenate([W_time, W_dis, W_speed], axis=0)
    t = time.reshape(-1).astype(jnp.int32)
    d = dis.reshape(-1).astype(jnp.int32)
    s = speed.reshape(-1).astype(jnp.int32)

    mesh = plsc.VectorSubcoreMesh(core_axis_name="c", subcore_axis_name="s")
    run = pl.kernel(
        _body,
        out_type=jax.ShapeDtypeStruct((3 * N, EMBED), jnp.float32),
        mesh=mesh,
        scratch_types=[
            pltpu.VMEM((CHUNK,), jnp.int32),
            pltpu.VMEM((CHUNK,), jnp.int32),
            pltpu.VMEM((CHUNK,), jnp.int32),
            pltpu.VMEM((3 * CHUNK,), jnp.int32),
            pltpu.VMEM((3 * CHUNK, EMBED), jnp.float32),
            pltpu.SemaphoreType.DMA,
        ],
        compiler_params=pltpu.CompilerParams(
            needs_layout_passes=False,
            use_tc_tiling_on_sc=False,
        ),
    )
    out = run(t, d, s, w_all)
    return out.reshape(B, L, 3 * EMBED)


# speedup vs baseline: 11.0927x; 1.0041x over previous
import functools

import jax
import jax.numpy as jnp
from jax import lax
from jax.experimental import pallas as pl
from jax.experimental.pallas import tpu as pltpu
from jax.experimental.pallas import tpu_sc as plsc

B, L = 4096, 200
EMBED = 8
OUT_D = 3 * EMBED

NC, NS = 2, 16
NW = NC * NS
B_PER_W = B // NW
R = 8
M = R * L
NSC = B_PER_W // R


def _body(t_hbm, d_hbm, s_hbm, wt_hbm, wd_hbm, ws_hbm, out_hbm,
          t_v, d_v, s_v, tb_v, db_v, sb_v, gsem, osem):
    wid = lax.axis_index("s") * NC + lax.axis_index("c")
    b0 = wid * B_PER_W

    def stage(p):
        off = (b0 + p * R) * L
        pltpu.sync_copy(t_hbm.at[pl.ds(off, M)], t_v.at[p % 2])
        pltpu.sync_copy(d_hbm.at[pl.ds(off, M)], d_v.at[p % 2])
        pltpu.sync_copy(s_hbm.at[pl.ds(off, M)], s_v.at[p % 2])

    stage(0)
    for p in range(NSC):
        row0 = b0 + p * R
        k = p % 2
        if p >= 2:
            for buf in (tb_v, db_v, sb_v):
                pltpu.make_async_copy(
                    wd_hbm.at[pl.ds(0, M)], buf.at[k], osem).wait()

        cps = [
            pltpu.async_copy(wt_hbm.at[t_v.at[k]], tb_v.at[k], gsem),
            pltpu.async_copy(wd_hbm.at[d_v.at[k]], db_v.at[k], gsem),
            pltpu.async_copy(ws_hbm.at[s_v.at[k]], sb_v.at[k], gsem),
        ]
        if p + 1 < NSC:
            stage(p + 1)
        for cp in cps:
            cp.wait()

        def write_row(r, _, k=k, row0=row0):
            for c, buf in enumerate((tb_v, db_v, sb_v)):
                pltpu.async_copy(
                    buf.at[k].at[pl.ds(r * L, L)],
                    out_hbm.at[row0 + r, :, pl.ds(c * EMBED, EMBED)],
                    osem,
                )
            return 0

        lax.fori_loop(0, R, write_row, 0)

    for k in range(2):
        for buf in (tb_v, db_v, sb_v):
            pltpu.make_async_copy(
                wd_hbm.at[pl.ds(0, M)], buf.at[k], osem).wait()


@functools.partial(jax.jit, static_argnames=())
def kernel(time, dis, speed, W_time, W_dis, W_speed):
    t = time.reshape(-1).astype(jnp.int32)
    d = dis.reshape(-1).astype(jnp.int32)
    s = speed.reshape(-1).astype(jnp.int32)

    mesh = plsc.VectorSubcoreMesh(core_axis_name="c", subcore_axis_name="s")
    run = pl.kernel(
        _body,
        out_type=jax.ShapeDtypeStruct((B, L, OUT_D), jnp.float32),
        mesh=mesh,
        scratch_types=[
            pltpu.VMEM((2, M), jnp.int32),
            pltpu.VMEM((2, M), jnp.int32),
            pltpu.VMEM((2, M), jnp.int32),
            pltpu.VMEM((2, M, EMBED), jnp.float32),
            pltpu.VMEM((2, M, EMBED), jnp.float32),
            pltpu.VMEM((2, M, EMBED), jnp.float32),
            pltpu.SemaphoreType.DMA,
            pltpu.SemaphoreType.DMA,
        ],
        compiler_params=pltpu.CompilerParams(
            needs_layout_passes=False,
            use_tc_tiling_on_sc=False,
        ),
    )
    return run(t, d, s, W_time, W_dis, W_speed)


# speedup vs baseline: 27.1960x; 2.4517x over previous
import functools

import jax
import jax.numpy as jnp
from jax import lax
from jax.experimental import pallas as pl
from jax.experimental.pallas import tpu as pltpu
from jax.experimental.pallas import tpu_sc as plsc

B, L = 4096, 200
EMBED = 8
OUT_D = 3 * EMBED

NC, NS = 2, 16
NW = NC * NS
BT = B // NW
LC = 8
M = LC * BT
NCH = L // LC


def _body(t_hbm, d_hbm, s_hbm, wt_hbm, wd_hbm, ws_hbm, out_hbm,
          ti_v, di_v, si_v, gb_v, ob_v, ssem, gsem, osem):
    wid = lax.axis_index("s") * NC + lax.axis_index("c")
    b0 = wid * BT
    iota = lax.iota(jnp.int32, 16)
    seconst = [jnp.full((16,), se, jnp.int32) for se in range(EMBED)]

    def stage(p, k):
        for idx_hbm, idx_v in ((t_hbm, ti_v), (d_hbm, di_v), (s_hbm, si_v)):
            for lp in range(LC):
                pltpu.async_copy(
                    idx_hbm.at[p * LC + lp, pl.ds(b0, BT)],
                    idx_v.at[k, pl.ds(lp * BT, BT)], ssem)

    def stage_wait(k):
        for idx_v in (ti_v, di_v, si_v):
            pltpu.make_async_copy(
                t_hbm.at[0, pl.ds(0, M)], idx_v.at[k], ssem).wait()

    def gathers(p, k):
        for c, (w_hbm, idx_v) in enumerate(
                ((wt_hbm, ti_v), (wd_hbm, di_v), (ws_hbm, si_v))):
            pltpu.async_copy(w_hbm.at[idx_v.at[k]], gb_v.at[k, c], gsem)

    def gathers_wait(k):
        for c in range(3):
            pltpu.make_async_copy(
                wd_hbm.at[ti_v.at[k]], gb_v.at[k, c], gsem).wait()

    def transpose(p, k):
        def per_l(lp, _):
            for c in range(3):
                src = gb_v.at[k, c]
                for se in range(EMBED):
                    for sbg in range(BT // 16):
                        rows = iota + (lp * BT + sbg * 16)
                        v = plsc.load_gather(src, [rows, seconst[se]])
                        ob_v[k, lp, c, se, pl.ds(sbg * 16, 16)] = v
            return 0
        lax.fori_loop(0, LC, per_l, 0)

    def write(p, k):
        pltpu.async_copy(
            ob_v.at[k], out_hbm.at[pl.ds(p * LC, LC), :, wid], osem)

    def write_wait(k):
        pltpu.make_async_copy(
            ob_v.at[k], out_hbm.at[pl.ds(0, LC), :, wid], osem).wait()

    stage(0, 0)
    stage(1, 1)
    stage_wait(0)
    gathers(0, 0)

    def chunk_body(p, _):
        for kk in (0, 1):

            @pl.when(lax.rem(p, 2) == kk)
            def _(kk=kk):
                @pl.when(p + 1 < NCH)
                def _():
                    stage_wait(kk ^ 1)

                gathers_wait(kk)

                @pl.when(p + 2 < NCH)
                def _():
                    stage(p + 2, kk)

                @pl.when(p + 1 < NCH)
                def _():
                    gathers(p + 1, kk ^ 1)

                @pl.when(p >= 2)
                def _():
                    write_wait(kk)

                transpose(p, kk)
                write(p, kk)

        return 0

    lax.fori_loop(0, NCH, chunk_body, 0)
    write_wait(0 if NCH % 2 == 1 else 1)
    write_wait(1 if NCH % 2 == 1 else 0)


@functools.partial(jax.jit, static_argnames=())
def kernel(time, dis, speed, W_time, W_dis, W_speed):
    t = time.T.astype(jnp.int32)
    d = dis.T.astype(jnp.int32)
    s = speed.T.astype(jnp.int32)

    mesh = plsc.VectorSubcoreMesh(core_axis_name="c", subcore_axis_name="s")
    run = pl.kernel(
        _body,
        out_type=jax.ShapeDtypeStruct((L, 3, NW, EMBED, BT), jnp.float32),
        mesh=mesh,
        scratch_types=[
            pltpu.VMEM((2, M), jnp.int32),
            pltpu.VMEM((2, M), jnp.int32),
            pltpu.VMEM((2, M), jnp.int32),
            pltpu.VMEM((2, 3, M, EMBED), jnp.float32),
            pltpu.VMEM((2, LC, 3, EMBED, BT), jnp.float32),
            pltpu.SemaphoreType.DMA,
            pltpu.SemaphoreType.DMA,
            pltpu.SemaphoreType.DMA,
        ],
        compiler_params=pltpu.CompilerParams(
            needs_layout_passes=False,
            use_tc_tiling_on_sc=False,
        ),
    )
    out5 = run(t, d, s, W_time, W_dis, W_speed)
    return out5.transpose(2, 4, 0, 1, 3).reshape(B, L, OUT_D)


# speedup vs baseline: 29.8953x; 1.0993x over previous
import functools

import jax
import jax.numpy as jnp
from jax import lax
from jax.experimental import pallas as pl
from jax.experimental.pallas import tpu as pltpu
from jax.experimental.pallas import tpu_sc as plsc

B, L = 4096, 200
EMBED = 8
OUT_D = 3 * EMBED

NC, NS = 2, 16
NW = NC * NS
BT = B // NW
LC = 8
M = LC * BT
NCH = L // LC
TIME_V = 1442
DIS_V = 100000
SPEED_V = 1002


def _body(t_hbm, d_hbm, s_hbm, wtT_hbm, wd_hbm, wsT_hbm, out_hbm,
          ti_v, di_v, si_v, gb_v, ob_v, wtT_v, wsT_v, ssem, gsem, osem):
    sid = lax.axis_index("s")
    wid = sid * NC + lax.axis_index("c")
    b0 = wid * BT
    iota = lax.iota(jnp.int32, 16)
    seconst = [jnp.full((16,), se, jnp.int32) for se in range(EMBED)]

    pltpu.sync_copy(wtT_hbm, wtT_v)
    pltpu.sync_copy(wsT_hbm, wsT_v)

    def stage(p, k):
        for idx_hbm, idx_v in ((t_hbm, ti_v), (d_hbm, di_v), (s_hbm, si_v)):
            for lp in range(LC):
                pltpu.async_copy(
                    idx_hbm.at[p * LC + lp, pl.ds(b0, BT)],
                    idx_v.at[k, pl.ds(lp * BT, BT)], ssem)

    def stage_wait(k):
        for idx_v in (ti_v, di_v, si_v):
            pltpu.make_async_copy(
                t_hbm.at[0, pl.ds(0, M)], idx_v.at[k], ssem).wait()

    def gathers(p, k):
        pltpu.async_copy(wd_hbm.at[di_v.at[k]], gb_v.at[k], gsem)

    def gathers_wait(k):
        pltpu.make_async_copy(
            wd_hbm.at[di_v.at[k]], gb_v.at[k], gsem).wait()

    def transpose(p, k):
        def per_l(lp, _):
            for sbg in range(BT // 16):
                base = lp * BT + sbg * 16
                tvi = ti_v[k, pl.ds(base, 16)]
                svi = si_v[k, pl.ds(base, 16)]
                rows = iota + base
                for se in range(EMBED):
                    vt = plsc.load_gather(wtT_v, [seconst[se], tvi])
                    ob_v[k, lp, 0, se, pl.ds(sbg * 16, 16)] = vt
                    vd = plsc.load_gather(gb_v.at[k], [rows, seconst[se]])
                    ob_v[k, lp, 1, se, pl.ds(sbg * 16, 16)] = vd
                    vs = plsc.load_gather(wsT_v, [seconst[se], svi])
                    ob_v[k, lp, 2, se, pl.ds(sbg * 16, 16)] = vs
            return 0
        lax.fori_loop(0, LC, per_l, 0)

    def write(p, k):
        pltpu.async_copy(
            ob_v.at[k], out_hbm.at[pl.ds(p * LC, LC), :, wid], osem)

    def write_wait(k):
        pltpu.make_async_copy(
            ob_v.at[k], out_hbm.at[pl.ds(0, LC), :, wid], osem).wait()

    stage(0, 0)
    stage(1, 1)
    stage_wait(0)
    gathers(0, 0)

    def chunk_body(p, _):
        for kk in (0, 1):

            @pl.when(lax.rem(p, 2) == kk)
            def _(kk=kk):
                @pl.when(p + 1 < NCH)
                def _():
                    stage_wait(kk ^ 1)

                gathers_wait(kk)

                @pl.when(p + 2 < NCH)
                def _():
                    stage(p + 2, kk)

                @pl.when(p + 1 < NCH)
                def _():
                    gathers(p + 1, kk ^ 1)

                @pl.when(p >= 2)
                def _():
                    write_wait(kk)

                transpose(p, kk)
                write(p, kk)

        return 0

    lax.fori_loop(0, NCH, chunk_body, 0)
    write_wait(0 if NCH % 2 == 1 else 1)
    write_wait(1 if NCH % 2 == 1 else 0)


@functools.partial(jax.jit, static_argnames=())
def kernel(time, dis, speed, W_time, W_dis, W_speed):
    t = time.T.astype(jnp.int32)
    d = dis.T.astype(jnp.int32)
    s = speed.T.astype(jnp.int32)

    mesh = plsc.VectorSubcoreMesh(core_axis_name="c", subcore_axis_name="s")
    run = pl.kernel(
        _body,
        out_type=jax.ShapeDtypeStruct((L, 3, NW, EMBED, BT), jnp.float32),
        mesh=mesh,
        scratch_types=[
            pltpu.VMEM((2, M), jnp.int32),
            pltpu.VMEM((2, M), jnp.int32),
            pltpu.VMEM((2, M), jnp.int32),
            pltpu.VMEM((2, M, EMBED), jnp.float32),
            pltpu.VMEM((2, LC, 3, EMBED, BT), jnp.float32),
            pltpu.VMEM((EMBED, TIME_V), jnp.float32),
            pltpu.VMEM((EMBED, SPEED_V), jnp.float32),
            pltpu.SemaphoreType.DMA,
            pltpu.SemaphoreType.DMA,
            pltpu.SemaphoreType.DMA,
        ],
        compiler_params=pltpu.CompilerParams(
            needs_layout_passes=False,
            use_tc_tiling_on_sc=False,
        ),
    )
    out5 = run(t, d, s, W_time.T, W_dis, W_speed.T)
    return out5.transpose(2, 4, 0, 1, 3).reshape(B, L, OUT_D)
